# baseline (speedup 1.0000x reference)
import functools

import jax
import jax.numpy as jnp
from jax import lax
from jax.experimental import pallas as pl
from jax.experimental.pallas import tpu as pltpu
from jax.experimental.pallas import tpu_sc as plsc

_K = 5
_CHUNK = 2048
_NC = 2
_NS = 16
_NW = _NC * _NS


def _topk_body(x_ref, idx_ref):
    B, D = x_ref.shape
    nch = D // _CHUNK
    sels = []
    for k in range(_K):
        best_v = jnp.full((B, 1), -jnp.inf, dtype=jnp.float32)
        best_i = jnp.zeros((B, 1), dtype=jnp.int32)
        for c in range(nch):
            v = x_ref[:, c * _CHUNK:(c + 1) * _CHUNK]
            col = jax.lax.broadcasted_iota(jnp.int32, (B, _CHUNK), 1) + c * _CHUNK
            for j in range(k):
                v = jnp.where(col == sels[j], -jnp.inf, v)
            cm = jnp.max(v, axis=1, keepdims=True)
            ci = jnp.min(jnp.where(v == cm, col, D), axis=1, keepdims=True)
            upd = cm > best_v
            best_v = jnp.where(upd, cm, best_v)
            best_i = jnp.where(upd, ci, best_i)
        sels.append(best_i)
    idx_ref[...] = jnp.concatenate(sels, axis=1)


def _sc_fill_body(D, ppw, idx_hbm, zsrc_hbm, out_hbm, zbuf_a, zbuf_b, idxbuf,
                  sem_a, sem_b):
    wid = lax.axis_index("s") * _NC + lax.axis_index("c")
    base_plane = wid * ppw

    pltpu.sync_copy(zsrc_hbm, zbuf_a)
    pltpu.sync_copy(zsrc_hbm, zbuf_b)

    abase = (base_plane // 8) * 8
    pltpu.sync_copy(idx_hbm.at[pl.ds(abase, 32)], idxbuf)
    off = base_plane - abase

    iota = lax.iota(jnp.int32, 16)
    c0 = idxbuf[pl.ds(0, 16)]
    c1 = idxbuf[pl.ds(16, 16)]
    poss = []
    for j in range(ppw):
        l = off + j
        poss.append(jnp.maximum(
            jnp.max(jnp.where(iota == l, c0, -1)),
            jnp.max(jnp.where(iota == l - 16, c1, -1)),
        ))

    bufs = (zbuf_a, zbuf_b)
    sems = (sem_a, sem_b)
    ones16 = jnp.ones((16,), jnp.float32)
    zeros16 = jnp.zeros((16,), jnp.float32)
    m0 = iota == 0
    handles = [None] * ppw
    for j in range(ppw):
        buf = bufs[j % 2]
        if j >= 2:
            handles[j - 2].wait()
            plsc.store_scatter(buf, [jnp.broadcast_to(poss[j - 2], (16,))],
                               zeros16, mask=m0)
        plsc.store_scatter(buf, [jnp.broadcast_to(poss[j], (16,))],
                           ones16, mask=m0)
        p = base_plane + j
        handles[j] = pltpu.async_copy(buf, out_hbm.at[p // _K, p % _K], sems[j % 2])
    handles[ppw - 2].wait()
    handles[ppw - 1].wait()


def kernel(similarities):
    B, D = similarities.shape
    idx = pl.pallas_call(
        _topk_body,
        out_shape=jax.ShapeDtypeStruct((B, _K), jnp.int32),
    )(similarities)

    nplanes = B * _K
    ppw = nplanes // _NW
    idx_pad = jnp.concatenate(
        [idx.reshape(-1), jnp.zeros((64,), jnp.int32)])
    zsrc = jnp.zeros((D,), jnp.float32)

    sc_fill = functools.partial(
        pl.kernel,
        out_type=jax.ShapeDtypeStruct((B, _K, D), jnp.float32),
        mesh=plsc.VectorSubcoreMesh(core_axis_name="c", subcore_axis_name="s"),
        compiler_params=pltpu.CompilerParams(needs_layout_passes=False),
        scratch_types=[
            pltpu.VMEM((D,), jnp.float32),
            pltpu.VMEM((D,), jnp.float32),
            pltpu.VMEM((32,), jnp.int32),
            pltpu.SemaphoreType.DMA,
            pltpu.SemaphoreType.DMA,
        ],
    )(functools.partial(_sc_fill_body, D, ppw))

    out = sc_fill(idx_pad, zsrc)
    return idx, out


# speedup vs baseline: 1.0201x; 1.0201x over previous
import functools

import jax
import jax.numpy as jnp
from jax import lax
from jax.experimental import pallas as pl
from jax.experimental.pallas import tpu as pltpu
from jax.experimental.pallas import tpu_sc as plsc

_K = 5
_CHUNK = 2048
_NC = 2
_NS = 16
_NW = _NC * _NS
_NSLOTS = 8


def _topk_body(x_ref, idx_ref):
    B, D = x_ref.shape
    nch = D // _CHUNK
    sels = []
    for k in range(_K):
        best_v = jnp.full((B, 1), -jnp.inf, dtype=jnp.float32)
        best_i = jnp.zeros((B, 1), dtype=jnp.int32)
        for c in range(nch):
            v = x_ref[:, c * _CHUNK:(c + 1) * _CHUNK]
            col = jax.lax.broadcasted_iota(jnp.int32, (B, _CHUNK), 1) + c * _CHUNK
            for j in range(k):
                v = jnp.where(col == sels[j], -jnp.inf, v)
            cm = jnp.max(v, axis=1, keepdims=True)
            ci = jnp.min(jnp.where(v == cm, col, D), axis=1, keepdims=True)
            upd = cm > best_v
            best_v = jnp.where(upd, cm, best_v)
            best_i = jnp.where(upd, ci, best_i)
        sels.append(best_i)
    idx_ref[...] = jnp.concatenate(sels, axis=1)


def _sc_zero_body(ppw, zsrc_hbm, out_hbm, zbuf, sem_z):
    wid = lax.axis_index("s") * _NC + lax.axis_index("c")
    base_plane = wid * ppw
    pltpu.sync_copy(zsrc_hbm, zbuf)
    handles = []
    for j in range(ppw):
        p = base_plane + j
        handles.append(pltpu.async_copy(zbuf, out_hbm.at[p // _K, p % _K], sem_z))
    for h in handles:
        h.wait()


def _patch_body(idx_ref, z_ref, out_ref, obuf, sems):
    B, K, D = out_ref.shape
    del z_ref
    lane = jax.lax.broadcasted_iota(jnp.int32, (1, 128), 1)
    for p in range(B * K):
        b, k = p // K, p % K
        v = idx_ref[b, k]
        seg = (v // 128) * 128
        slot = p % _NSLOTS
        if p >= _NSLOTS:
            pltpu.make_async_copy(
                obuf.at[pl.ds(slot, 1)], out_ref.at[0, pl.ds(0, 1), pl.ds(0, 128)], sems.at[slot]
            ).wait()
        obuf[pl.ds(slot, 1), :] = jnp.where(lane == (v - seg), 1.0, 0.0).astype(jnp.float32)
        pltpu.make_async_copy(
            obuf.at[pl.ds(slot, 1)], out_ref.at[b, pl.ds(k, 1), pl.ds(seg, 128)], sems.at[slot]
        ).start()
    for slot in range(_NSLOTS):
        pltpu.make_async_copy(
            obuf.at[pl.ds(slot, 1)], out_ref.at[0, pl.ds(0, 1), pl.ds(0, 128)], sems.at[slot]
        ).wait()


def kernel(similarities):
    B, D = similarities.shape
    nplanes = B * _K
    ppw = nplanes // _NW
    zsrc = jnp.zeros((D,), jnp.float32)

    sc_zero = functools.partial(
        pl.kernel,
        out_type=jax.ShapeDtypeStruct((B, _K, D), jnp.float32),
        mesh=plsc.VectorSubcoreMesh(core_axis_name="c", subcore_axis_name="s"),
        scratch_types=[
            pltpu.VMEM((D,), jnp.float32),
            pltpu.SemaphoreType.DMA,
        ],
    )(functools.partial(_sc_zero_body, ppw))
    z0 = sc_zero(zsrc)

    idx = pl.pallas_call(
        _topk_body,
        out_shape=jax.ShapeDtypeStruct((B, _K), jnp.int32),
    )(similarities)

    out = pl.pallas_call(
        _patch_body,
        in_specs=[
            pl.BlockSpec(memory_space=pltpu.SMEM),
            pl.BlockSpec(memory_space=pl.ANY),
        ],
        out_specs=pl.BlockSpec(memory_space=pl.ANY),
        out_shape=jax.ShapeDtypeStruct((B, _K, D), jnp.float32),
        scratch_shapes=[
            pltpu.VMEM((_NSLOTS, 128), jnp.float32),
            pltpu.SemaphoreType.DMA((_NSLOTS,)),
        ],
        input_output_aliases={1: 0},
    )(idx, z0)
    return idx, out


# speedup vs baseline: 1.2601x; 1.2354x over previous
import jax
import jax.numpy as jnp
from jax.experimental import pallas as pl
from jax.experimental.pallas import tpu as pltpu

_K = 5
_CHUNK = 2048
_DBLK = 8192


def _topk_body(x_ref, idx_ref):
    B, D = x_ref.shape
    nch = D // _CHUNK
    sels = []
    for k in range(_K):
        best_v = jnp.full((B, 1), -jnp.inf, dtype=jnp.float32)
        best_i = jnp.zeros((B, 1), dtype=jnp.int32)
        for c in range(nch):
            v = x_ref[:, c * _CHUNK:(c + 1) * _CHUNK]
            col = jax.lax.broadcasted_iota(jnp.int32, (B, _CHUNK), 1) + c * _CHUNK
            for j in range(k):
                v = jnp.where(col == sels[j], -jnp.inf, v)
            cm = jnp.max(v, axis=1, keepdims=True)
            ci = jnp.min(jnp.where(v == cm, col, D), axis=1, keepdims=True)
            upd = cm > best_v
            best_v = jnp.where(upd, cm, best_v)
            best_i = jnp.where(upd, ci, best_i)
        sels.append(best_i)
    idx_ref[...] = jnp.concatenate(sels, axis=1)


def _onehot_body(idx_ref, out_ref):
    i = pl.program_id(0)
    B, K, dblk = out_ref.shape
    idxv = idx_ref[...][:, :, None]
    col = jax.lax.broadcasted_iota(jnp.int32, (B, K, dblk), 2) + i * dblk
    out_ref[...] = jnp.where(col == idxv, 1.0, 0.0).astype(jnp.float32)


def kernel(similarities):
    B, D = similarities.shape
    idx = pl.pallas_call(
        _topk_body,
        out_shape=jax.ShapeDtypeStruct((B, _K), jnp.int32),
    )(similarities)

    oh = pl.pallas_call(
        _onehot_body,
        grid=(D // _DBLK,),
        in_specs=[pl.BlockSpec((B, _K), lambda i: (0, 0))],
        out_specs=pl.BlockSpec((B, _K, _DBLK), lambda i: (0, 0, i)),
        out_shape=jax.ShapeDtypeStruct((B, _K, D), jnp.float32),
        compiler_params=pltpu.CompilerParams(
            dimension_semantics=("arbitrary",),
        ),
    )(idx)
    return idx, oh
